# baseline (speedup 1.0000x reference)
import functools

import jax
import jax.numpy as jnp
from jax import lax
from jax.experimental import pallas as pl
from jax.experimental.pallas import tpu as pltpu
from jax.experimental.pallas import tpu_sc as plsc

D = 64
S = 50
BT = 16384
NBH = BT // 128
NC, NS = 2, 16
NW = NC * NS
BH_PER_W = NBH // NW
CW = BH_PER_W * 128

_mesh = plsc.VectorSubcoreMesh(core_axis_name="c", subcore_axis_name="s")


@functools.partial(
    pl.kernel,
    mesh=_mesh,
    out_type=jax.ShapeDtypeStruct((S, 8, NBH, 1024), jnp.float32),
    compiler_params=pltpu.CompilerParams(
        use_tc_tiling_on_sc=False, needs_layout_passes=False),
    scratch_types=[
        pltpu.VMEM((S, CW), jnp.int32),
        pltpu.VMEM((2, CW, D), jnp.float32),
        pltpu.VMEM((4, 8 * 1024), jnp.float32),
        pltpu.SemaphoreType.DMA,
        pltpu.SemaphoreType.DMA,
    ],
)
def _gather_kernel(idx_hbm, table_hbm, out_hbm, idx_v, rows_v, t_v,
                   sem_g, sem_w):
    wid = lax.axis_index("s") * NC + lax.axis_index("c")

    pltpu.sync_copy(idx_hbm.at[:, pl.ds(wid * CW, CW)], idx_v)

    def launch_gather(s, buf):
        pltpu.async_copy(
            table_hbm.at[idx_v.at[s, pl.ds(0, CW)]],
            rows_v.at[buf],
            sem_g,
        )

    def wait_gather(buf):
        pltpu.make_async_copy(
            table_hbm.at[pl.ds(0, CW)], rows_v.at[buf], sem_g
        ).wait()

    def launch_writes(s, ub, tb):
        for jh in range(8):
            pltpu.async_copy(
                t_v.at[tb, pl.ds(jh * 1024, 1024)],
                out_hbm.at[s, jh, wid * BH_PER_W + ub],
                sem_w,
            )

    def wait_writes(tb):
        for jh in range(8):
            pltpu.make_async_copy(
                t_v.at[tb, pl.ds(jh * 1024, 1024)], out_hbm.at[0, 0, 0], sem_w
            ).wait()

    lane = lax.iota(jnp.int32, 16)
    sixteen = jnp.full((16,), 16, jnp.int32)

    def transpose_half(buf, half):
        @pl.loop(0, D)
        def _feat(j):
            jbase = j * 128
            col = jnp.full((16,), 0, jnp.int32) + j
            rowvec = lane + half * 256
            for ubl in range(2):
                tb = half * 2 + ubl
                for bq in range(8):
                    vec = plsc.load_gather(rows_v.at[buf], [rowvec, col])
                    t_v[tb, pl.ds(jbase + bq * 16, 16)] = vec
                    rowvec = rowvec + sixteen

    launch_gather(0, 0)
    launch_gather(1, 1)

    @pl.loop(0, S, step=2)
    def _chunks(s0):
        for b in range(2):
            s = s0 + b
            wait_gather(b)
            for half in range(2):
                @pl.when(s >= 1)
                def _():
                    wait_writes(half * 2)
                    wait_writes(half * 2 + 1)

                transpose_half(b, half)
                for ubl in range(2):
                    launch_writes(s, half * 2 + ubl, half * 2 + ubl)

            @pl.when(s + 2 < S)
            def _():
                launch_gather(s + 2, b)

    for tb in range(4):
        wait_writes(tb)


def kernel(inputs, weight):
    out5 = _gather_kernel(inputs.T, weight)
    return (out5.reshape(S, 8, NBH, 8, 128)
            .transpose(2, 4, 0, 1, 3)
            .reshape(BT, S, D))


# speedup vs baseline: 1.1052x; 1.1052x over previous
import functools

import jax
import jax.numpy as jnp
from jax import lax
from jax.experimental import pallas as pl
from jax.experimental.pallas import tpu as pltpu
from jax.experimental.pallas import tpu_sc as plsc

D = 64
S = 50
BT = 16384
NBH = BT // 128
NC, NS = 2, 16
NW = NC * NS
BH_PER_W = NBH // NW
CW = BH_PER_W * 128
TWORDS = BH_PER_W * 8 * 1024

_mesh = plsc.VectorSubcoreMesh(core_axis_name="c", subcore_axis_name="s")


@functools.partial(
    pl.kernel,
    mesh=_mesh,
    out_type=jax.ShapeDtypeStruct((S, 8, NBH, 1024), jnp.float32),
    compiler_params=pltpu.CompilerParams(
        use_tc_tiling_on_sc=False, needs_layout_passes=False),
    scratch_types=[
        pltpu.VMEM((S, CW), jnp.int32),
        pltpu.VMEM((2, CW, D), jnp.float32),
        pltpu.VMEM((TWORDS + 128,), jnp.float32),
        pltpu.SemaphoreType.DMA,
        pltpu.SemaphoreType.DMA,
    ],
)
def _gather_kernel(idx_hbm, table_hbm, out_hbm, idx_v, rows_v, t_v,
                   sem_g, sem_w):
    wid = lax.axis_index("s") * NC + lax.axis_index("c")

    pltpu.sync_copy(idx_hbm.at[:, pl.ds(wid * CW, CW)], idx_v)

    def launch_gather(s, buf):
        pltpu.async_copy(
            table_hbm.at[idx_v.at[s, pl.ds(0, CW)]],
            rows_v.at[buf],
            sem_g,
        )

    def wait_gather(buf):
        pltpu.make_async_copy(
            table_hbm.at[pl.ds(0, CW)], rows_v.at[buf], sem_g
        ).wait()

    def launch_writes(s):
        for ub in range(BH_PER_W):
            for jh in range(8):
                pltpu.async_copy(
                    t_v.at[pl.ds(ub * 8192 + jh * 1024, 1024)],
                    out_hbm.at[s, jh, wid * BH_PER_W + ub],
                    sem_w,
                )

    def wait_writes():
        for _ in range(BH_PER_W * 8):
            pltpu.make_async_copy(
                t_v.at[pl.ds(0, 1024)], out_hbm.at[0, 0, 0], sem_w
            ).wait()

    lane = lax.iota(jnp.int32, 16)
    jvecs = [(jq * 16 + lane) * 128 for jq in range(4)]

    RUNROLL = 8

    def transpose_chunk(buf):
        @pl.loop(0, CW, step=RUNROLL)
        def _rows(r0):
            for k in range(RUNROLL):
                r = r0 + k
                ub = r // 128
                bl = r % 128
                dst = t_v.at[pl.ds(ub * 8192, 8192 + 128)]
                for jq in range(4):
                    vec = rows_v[buf, r, pl.ds(jq * 16, 16)]
                    plsc.store_scatter(dst, [jvecs[jq] + bl], vec)

    launch_gather(0, 0)
    launch_gather(1, 1)

    @pl.loop(0, S, step=2)
    def _chunks(s0):
        for b in range(2):
            s = s0 + b
            wait_gather(b)

            @pl.when(s >= 1)
            def _():
                wait_writes()

            transpose_chunk(b)
            launch_writes(s)

            @pl.when(s + 2 < S)
            def _():
                launch_gather(s + 2, b)

    wait_writes()


def kernel(inputs, weight):
    out5 = _gather_kernel(inputs.T, weight)
    return (out5.reshape(S, 8, NBH, 8, 128)
            .transpose(2, 4, 0, 1, 3)
            .reshape(BT, S, D))


# speedup vs baseline: 1.1170x; 1.0107x over previous
import functools

import jax
import jax.numpy as jnp
from jax import lax
from jax.experimental import pallas as pl
from jax.experimental.pallas import tpu as pltpu
from jax.experimental.pallas import tpu_sc as plsc

D = 64
S = 50
BT = 16384
NBH = BT // 128
NC, NS = 2, 16
NW = NC * NS
BH_PER_W = NBH // NW
CW = BH_PER_W * 128
TWORDS = BH_PER_W * 8 * 1024

_mesh = plsc.VectorSubcoreMesh(core_axis_name="c", subcore_axis_name="s")


@functools.partial(
    pl.kernel,
    mesh=_mesh,
    out_type=jax.ShapeDtypeStruct((S, 8, NBH, 1024), jnp.float32),
    compiler_params=pltpu.CompilerParams(
        use_tc_tiling_on_sc=False, needs_layout_passes=False,
        disable_bounds_checks=True),
    scratch_types=[
        pltpu.VMEM((S, CW), jnp.int32),
        pltpu.VMEM((2, CW, D), jnp.float32),
        pltpu.VMEM((TWORDS + 128,), jnp.float32),
        pltpu.SemaphoreType.DMA,
        pltpu.SemaphoreType.DMA,
    ],
)
def _gather_kernel(idx_hbm, table_hbm, out_hbm, idx_v, rows_v, t_v,
                   sem_g, sem_w):
    wid = lax.axis_index("s") * NC + lax.axis_index("c")

    pltpu.sync_copy(idx_hbm.at[:, pl.ds(wid * CW, CW)], idx_v)

    def launch_gather(s, buf):
        pltpu.async_copy(
            table_hbm.at[idx_v.at[s, pl.ds(0, CW)]],
            rows_v.at[buf],
            sem_g,
        )

    def wait_gather(buf):
        pltpu.make_async_copy(
            table_hbm.at[pl.ds(0, CW)], rows_v.at[buf], sem_g
        ).wait()

    def launch_writes(s):
        for ub in range(BH_PER_W):
            for jh in range(8):
                pltpu.async_copy(
                    t_v.at[pl.ds(ub * 8192 + jh * 1024, 1024)],
                    out_hbm.at[s, jh, wid * BH_PER_W + ub],
                    sem_w,
                )

    def wait_writes():
        for _ in range(BH_PER_W * 8):
            pltpu.make_async_copy(
                t_v.at[pl.ds(0, 1024)], out_hbm.at[0, 0, 0], sem_w
            ).wait()

    lane = lax.iota(jnp.int32, 16)
    jvecs = [(jq * 16 + lane) * 128 for jq in range(4)]

    RUNROLL = 8

    def transpose_chunk(buf):
        @pl.loop(0, CW, step=RUNROLL)
        def _rows(r0):
            for k in range(RUNROLL):
                r = r0 + k
                ub = r // 128
                bl = r % 128
                dst = t_v.at[pl.ds(ub * 8192, 8192 + 128)]
                vecs = [rows_v[buf, r, pl.ds(jq * 16, 16)] for jq in range(4)]
                idxs = [jvecs[jq] + bl for jq in range(4)]
                for jq in range(4):
                    plsc.store_scatter(dst, [idxs[jq]], vecs[jq])

    launch_gather(0, 0)
    launch_gather(1, 1)

    @pl.loop(0, S, step=2)
    def _chunks(s0):
        for b in range(2):
            s = s0 + b
            wait_gather(b)

            @pl.when(s >= 1)
            def _():
                wait_writes()

            transpose_chunk(b)
            launch_writes(s)

            @pl.when(s + 2 < S)
            def _():
                launch_gather(s + 2, b)

    wait_writes()


def kernel(inputs, weight):
    out5 = _gather_kernel(inputs.T, weight)
    return (out5.reshape(S, 8, NBH, 8, 128)
            .transpose(2, 4, 0, 1, 3)
            .reshape(BT, S, D))


# speedup vs baseline: 2.0210x; 1.8093x over previous
import functools

import jax
import jax.numpy as jnp
from jax import lax
from jax.experimental import pallas as pl
from jax.experimental.pallas import tpu as pltpu
from jax.experimental.pallas import tpu_sc as plsc

D = 64
S = 50
BT = 16384
NBH = BT // 128
NC, NS = 2, 16
NW = NC * NS
BH_PER_W = NBH // NW
CW = 256
NCHUNK = S * 2
TPAD = 129

_mesh = plsc.VectorSubcoreMesh(core_axis_name="c", subcore_axis_name="s")


@functools.partial(
    pl.kernel,
    mesh=_mesh,
    out_type=jax.ShapeDtypeStruct((S, 8, NBH, 8, 128), jnp.float32),
    compiler_params=pltpu.CompilerParams(
        use_tc_tiling_on_sc=False, needs_layout_passes=False,
        disable_bounds_checks=True),
    scratch_types=[
        pltpu.VMEM((S, BH_PER_W * 128), jnp.int32),
        pltpu.VMEM((2, CW, D), jnp.float32),
        pltpu.VMEM((2, 2, D, TPAD), jnp.float32),
        pltpu.SemaphoreType.DMA,
        pltpu.SemaphoreType.DMA,
    ],
)
def _gather_kernel(idx_hbm, table_hbm, out_hbm, idx_v, rows_v, t_v,
                   sem_g, sem_w):
    wid = lax.axis_index("s") * NC + lax.axis_index("c")

    pltpu.sync_copy(idx_hbm.at[:, pl.ds(wid * (BH_PER_W * 128),
                                        BH_PER_W * 128)], idx_v)

    def launch_gather(c, buf):
        s = c // 2
        half = c % 2
        pltpu.async_copy(
            table_hbm.at[idx_v.at[s, pl.ds(half * CW, CW)]],
            rows_v.at[buf],
            sem_g,
        )

    def wait_gather(buf):
        pltpu.make_async_copy(
            table_hbm.at[pl.ds(0, CW)], rows_v.at[buf], sem_g
        ).wait()

    def launch_writes(c, tb):
        s = c // 2
        half = c % 2
        for ub in range(2):
            bh = wid * BH_PER_W + half * 2 + ub
            for jh in range(8):
                pltpu.async_copy(
                    t_v.at[tb, ub, pl.ds(jh * 8, 8), pl.ds(0, 128)],
                    out_hbm.at[s, jh, bh],
                    sem_w,
                )

    def wait_writes(tb):
        for _ in range(16):
            pltpu.make_async_copy(
                t_v.at[tb, 0, pl.ds(0, 8), pl.ds(0, 128)],
                out_hbm.at[0, 0, 0],
                sem_w,
            ).wait()

    lane = lax.iota(jnp.int32, 16)
    jrows = [jq * 16 + lane for jq in range(4)]

    RUNROLL = 8

    def transpose_chunk(buf, tb):
        @pl.loop(0, CW, step=RUNROLL)
        def _rows(r0):
            for k in range(RUNROLL):
                r = r0 + k
                ub = r // 128
                bl = r % 128
                dst = t_v.at[tb, ub]
                colv = jnp.full((16,), 0, jnp.int32) + bl
                vecs = [rows_v[buf, r, pl.ds(jq * 16, 16)] for jq in range(4)]
                for jq in range(4):
                    plsc.store_scatter(dst, [jrows[jq], colv], vecs[jq])

    launch_gather(0, 0)
    launch_gather(1, 1)

    @pl.loop(0, NCHUNK, step=2)
    def _chunks(c0):
        for b in range(2):
            c = c0 + b
            wait_gather(b)

            @pl.when(c >= 2)
            def _():
                wait_writes(b)

            transpose_chunk(b, b)
            launch_writes(c, b)

            @pl.when(c + 2 < NCHUNK)
            def _():
                launch_gather(c + 2, b)

    wait_writes(0)
    wait_writes(1)


def kernel(inputs, weight):
    out5 = _gather_kernel(inputs.T, weight)
    return out5.transpose(2, 4, 0, 1, 3).reshape(BT, S, D)
